# Initial kernel scaffold; baseline (speedup 1.0000x reference)
import jax
import jax.numpy as jnp
from jax.experimental import pallas as pl


def kernel(classifications, regressions, anchors, annotations):
    raise NotImplementedError("write your pallas kernel here")


# speedup vs baseline: 1.2628x; 1.2628x over previous
import functools

import jax
import jax.numpy as jnp
from jax import lax
from jax.experimental import pallas as pl

_BN = 2000


def _body(cls_ref, reg_ref, anc_ref, ann_ref, cls_o, npos_o, reg_o, *, bn, m, c):
    i = pl.program_id(1)

    @pl.when(i == 0)
    def _init():
        cls_o[...] = jnp.zeros_like(cls_o)
        npos_o[...] = jnp.zeros_like(npos_o)
        reg_o[...] = jnp.zeros_like(reg_o)

    ann = ann_ref[0]
    anc = anc_ref[0]
    ax1 = anc[:, 0:1]
    ay1 = anc[:, 1:2]
    ax2 = anc[:, 2:3]
    ay2 = anc[:, 3:4]
    bx1 = ann[0:1, :]
    by1 = ann[1:2, :]
    bx2 = ann[2:3, :]
    by2 = ann[3:4, :]
    blab = ann[4:5, :]

    iw = jnp.maximum(jnp.minimum(ax2, bx2) - jnp.maximum(ax1, bx1), 0.0)
    ih = jnp.maximum(jnp.minimum(ay2, by2) - jnp.maximum(ay1, by1), 0.0)
    inter = iw * ih
    area_a = (ax2 - ax1) * (ay2 - ay1)
    area_b = (bx2 - bx1) * (by2 - by1)
    union = jnp.maximum(area_a + area_b - inter, 1e-8)
    iou = inter / union
    valid = blab != -1.0
    iou = jnp.where(valid, iou, -1.0)

    iou_max = jnp.max(iou, axis=1, keepdims=True)
    mi = lax.broadcasted_iota(jnp.int32, (bn, m), 1)
    amax = jnp.min(jnp.where(iou == iou_max, mi, m), axis=1, keepdims=True)
    sel = (mi == amax).astype(jnp.float32)

    gx1 = jnp.sum(sel * bx1, axis=1, keepdims=True)
    gy1 = jnp.sum(sel * by1, axis=1, keepdims=True)
    gx2 = jnp.sum(sel * bx2, axis=1, keepdims=True)
    gy2 = jnp.sum(sel * by2, axis=1, keepdims=True)
    glab = jnp.sum(sel * blab, axis=1, keepdims=True)

    pos = iou_max >= 0.5
    keep = jnp.logical_or(iou_max < 0.4, pos)
    npos_part = jnp.sum(pos.astype(jnp.float32))

    cls = jnp.clip(cls_ref[0], 0.0001, 1.0 - 0.0001)
    neg_elem = (0.75 * (cls * cls)) * (-jnp.log(1.0 - cls))
    row_neg = jnp.sum(neg_elem, axis=1, keepdims=True)

    ci = lax.broadcasted_iota(jnp.int32, (bn, c), 1)
    glab_i = glab.astype(jnp.int32)
    g = jnp.sum(jnp.where(ci == glab_i, cls, 0.0), axis=1, keepdims=True)
    pos_e = (0.25 * ((1.0 - g) * (1.0 - g))) * (-jnp.log(g))
    neg_e = (0.75 * (g * g)) * (-jnp.log(1.0 - g))
    corr = jnp.where(pos, pos_e - neg_e, 0.0)
    cls_part = jnp.sum(jnp.where(keep, row_neg, 0.0)) + jnp.sum(corr)

    aw = ax2 - ax1
    ah = ay2 - ay1
    acx = ax1 + 0.5 * aw
    acy = ay1 + 0.5 * ah
    gw = gx2 - gx1
    gh = gy2 - gy1
    gcx = gx1 + 0.5 * gw
    gcy = gy1 + 0.5 * gh
    gw = jnp.maximum(gw, 1.0)
    gh = jnp.maximum(gh, 1.0)
    t0 = ((gcx - acx) / aw) / 0.1
    t1 = ((gcy - acy) / ah) / 0.1
    t2 = jnp.log(gw / aw) / 0.2
    t3 = jnp.log(gh / ah) / 0.2
    reg = reg_ref[0]
    d0 = jnp.abs(t0 - reg[:, 0:1])
    d1 = jnp.abs(t1 - reg[:, 1:2])
    d2 = jnp.abs(t2 - reg[:, 2:3])
    d3 = jnp.abs(t3 - reg[:, 3:4])

    def smooth_l1(d):
        return jnp.where(d < 1.0 / 9.0, 0.5 * 9.0 * (d * d), d - 0.5 / 9.0)

    rl = smooth_l1(d0) + smooth_l1(d1) + smooth_l1(d2) + smooth_l1(d3)
    reg_part = jnp.sum(jnp.where(pos, rl, 0.0))

    cls_o[...] += jnp.full(cls_o.shape, cls_part, jnp.float32)
    npos_o[...] += jnp.full(npos_o.shape, npos_part, jnp.float32)
    reg_o[...] += jnp.full(reg_o.shape, reg_part, jnp.float32)


@jax.jit
def kernel(classifications, regressions, anchors, annotations):
    b, n, c = classifications.shape
    m = annotations.shape[1]
    bn = _BN
    ann_t = annotations.transpose(0, 2, 1)

    body = functools.partial(_body, bn=bn, m=m, c=c)
    out_sds = jax.ShapeDtypeStruct((b, 1, 128), jnp.float32)
    cls_s, npos, reg_s = pl.pallas_call(
        body,
        grid=(b, n // bn),
        in_specs=[
            pl.BlockSpec((1, bn, c), lambda bb, ii: (bb, ii, 0)),
            pl.BlockSpec((1, bn, 4), lambda bb, ii: (bb, ii, 0)),
            pl.BlockSpec((1, bn, 4), lambda bb, ii: (bb, ii, 0)),
            pl.BlockSpec((1, 5, m), lambda bb, ii: (bb, 0, 0)),
        ],
        out_specs=[
            pl.BlockSpec((1, 1, 128), lambda bb, ii: (bb, 0, 0)),
            pl.BlockSpec((1, 1, 128), lambda bb, ii: (bb, 0, 0)),
            pl.BlockSpec((1, 1, 128), lambda bb, ii: (bb, 0, 0)),
        ],
        out_shape=[out_sds, out_sds, out_sds],
    )(classifications, regressions, anchors, ann_t)

    cls_s = cls_s[:, 0, 0]
    npos = npos[:, 0, 0]
    reg_s = reg_s[:, 0, 0]
    cls_tot = jnp.where(npos > 0, cls_s / jnp.maximum(npos, 1.0), 0.0)
    reg_tot = jnp.where(npos > 0, reg_s / jnp.maximum(4.0 * npos, 1.0), 0.0)
    return jnp.mean(cls_tot), jnp.mean(reg_tot)


# speedup vs baseline: 3.5790x; 2.8341x over previous
import functools

import jax
import jax.numpy as jnp
from jax import lax
from jax.experimental import pallas as pl

_BN = 2000


def _body(cls_ref, reg_ref, anc_ref, ann_ref, cls_o, npos_o, reg_o, *, bn, m, c):
    i = pl.program_id(1)

    @pl.when(i == 0)
    def _init():
        cls_o[...] = jnp.zeros_like(cls_o)
        npos_o[...] = jnp.zeros_like(npos_o)
        reg_o[...] = jnp.zeros_like(reg_o)

    ann = ann_ref[0]
    bx1 = ann[:, 0:1]
    by1 = ann[:, 1:2]
    bx2 = ann[:, 2:3]
    by2 = ann[:, 3:4]
    blab = ann[:, 4:5]
    anc = anc_ref[0, 0]
    ax1 = anc[0:1, :]
    ay1 = anc[1:2, :]
    ax2 = anc[2:3, :]
    ay2 = anc[3:4, :]

    iw = jnp.maximum(jnp.minimum(ax2, bx2) - jnp.maximum(ax1, bx1), 0.0)
    ih = jnp.maximum(jnp.minimum(ay2, by2) - jnp.maximum(ay1, by1), 0.0)
    inter = iw * ih
    area_a = (ax2 - ax1) * (ay2 - ay1)
    area_b = (bx2 - bx1) * (by2 - by1)
    union = jnp.maximum(area_a + area_b - inter, 1e-8)
    iou = inter / union
    iou = jnp.where(blab != -1.0, iou, -1.0)

    iou_max = jnp.max(iou, axis=0, keepdims=True)
    mi = lax.broadcasted_iota(jnp.int32, (m, bn), 0)
    amax = jnp.min(jnp.where(iou == iou_max, mi, m), axis=0, keepdims=True)
    sel = (mi == amax).astype(jnp.float32)

    gx1 = jnp.sum(sel * bx1, axis=0, keepdims=True)
    gy1 = jnp.sum(sel * by1, axis=0, keepdims=True)
    gx2 = jnp.sum(sel * bx2, axis=0, keepdims=True)
    gy2 = jnp.sum(sel * by2, axis=0, keepdims=True)
    glab = jnp.sum(sel * blab, axis=0, keepdims=True)

    pos = iou_max >= 0.5
    keep = jnp.logical_or(iou_max < 0.4, pos)
    npos_part = jnp.sum(pos.astype(jnp.float32))

    aw = ax2 - ax1
    ah = ay2 - ay1
    acx = ax1 + 0.5 * aw
    acy = ay1 + 0.5 * ah
    gw = gx2 - gx1
    gh = gy2 - gy1
    gcx = gx1 + 0.5 * gw
    gcy = gy1 + 0.5 * gh
    gw = jnp.maximum(gw, 1.0)
    gh = jnp.maximum(gh, 1.0)
    t0 = ((gcx - acx) / aw) / 0.1
    t1 = ((gcy - acy) / ah) / 0.1
    t2 = jnp.log(gw / aw) / 0.2
    t3 = jnp.log(gh / ah) / 0.2
    reg = reg_ref[0, 0]
    d0 = jnp.abs(t0 - reg[0:1, :])
    d1 = jnp.abs(t1 - reg[1:2, :])
    d2 = jnp.abs(t2 - reg[2:3, :])
    d3 = jnp.abs(t3 - reg[3:4, :])

    def smooth_l1(d):
        return jnp.where(d < 1.0 / 9.0, 0.5 * 9.0 * (d * d), d - 0.5 / 9.0)

    rl = smooth_l1(d0) + smooth_l1(d1) + smooth_l1(d2) + smooth_l1(d3)
    reg_part = jnp.sum(jnp.where(pos, rl, 0.0))

    keep_col = jnp.transpose(keep.astype(jnp.float32), (1, 0))
    glab_col = jnp.transpose(glab, (1, 0))

    cls = jnp.clip(cls_ref[0], 0.0001, 1.0 - 0.0001)
    logm = jnp.log(1.0 - cls)
    masked = ((0.75 * (cls * cls)) * logm) * keep_col
    ci = lax.broadcasted_iota(jnp.int32, (bn, c), 1)
    g_col = jnp.sum(jnp.where(ci == glab_col.astype(jnp.int32), cls, 0.0),
                    axis=1, keepdims=True)
    g = jnp.transpose(g_col, (1, 0))
    pos_e = (0.25 * ((1.0 - g) * (1.0 - g))) * (-jnp.log(g))
    neg_e = (0.75 * (g * g)) * (-jnp.log(1.0 - g))
    corr = jnp.where(pos, pos_e - neg_e, 0.0)
    cls_part = jnp.sum(corr) - jnp.sum(masked)

    cls_o[...] += jnp.full(cls_o.shape, cls_part, jnp.float32)
    npos_o[...] += jnp.full(npos_o.shape, npos_part, jnp.float32)
    reg_o[...] += jnp.full(reg_o.shape, reg_part, jnp.float32)


@jax.jit
def kernel(classifications, regressions, anchors, annotations):
    b, n, c = classifications.shape
    m = annotations.shape[1]
    bn = _BN
    nb = n // bn
    anc_t = anchors.reshape(b, nb, bn, 4).transpose(0, 1, 3, 2)
    reg_t = regressions.reshape(b, nb, bn, 4).transpose(0, 1, 3, 2)

    body = functools.partial(_body, bn=bn, m=m, c=c)
    out_sds = jax.ShapeDtypeStruct((b, 1, 128), jnp.float32)
    cls_s, npos, reg_s = pl.pallas_call(
        body,
        grid=(b, n // bn),
        in_specs=[
            pl.BlockSpec((1, bn, c), lambda bb, ii: (bb, ii, 0)),
            pl.BlockSpec((1, 1, 4, bn), lambda bb, ii: (bb, ii, 0, 0)),
            pl.BlockSpec((1, 1, 4, bn), lambda bb, ii: (bb, ii, 0, 0)),
            pl.BlockSpec((1, m, 5), lambda bb, ii: (bb, 0, 0)),
        ],
        out_specs=[
            pl.BlockSpec((1, 1, 128), lambda bb, ii: (bb, 0, 0)),
            pl.BlockSpec((1, 1, 128), lambda bb, ii: (bb, 0, 0)),
            pl.BlockSpec((1, 1, 128), lambda bb, ii: (bb, 0, 0)),
        ],
        out_shape=[out_sds, out_sds, out_sds],
    )(classifications, reg_t, anc_t, annotations)

    cls_s = cls_s[:, 0, 0]
    npos = npos[:, 0, 0]
    reg_s = reg_s[:, 0, 0]
    cls_tot = jnp.where(npos > 0, cls_s / jnp.maximum(npos, 1.0), 0.0)
    reg_tot = jnp.where(npos > 0, reg_s / jnp.maximum(4.0 * npos, 1.0), 0.0)
    return jnp.mean(cls_tot), jnp.mean(reg_tot)


# speedup vs baseline: 3.7577x; 1.0499x over previous
import functools

import jax
import jax.numpy as jnp
from jax import lax
from jax.experimental import pallas as pl

_BN = 4000


def _body(cls_ref, reg_ref, anc_ref, ann_ref, cls_o, npos_o, reg_o, *, bn, m, c):
    i = pl.program_id(1)

    @pl.when(i == 0)
    def _init():
        cls_o[...] = jnp.zeros_like(cls_o)
        npos_o[...] = jnp.zeros_like(npos_o)
        reg_o[...] = jnp.zeros_like(reg_o)

    ann = ann_ref[0]
    bx1 = ann[:, 0:1]
    by1 = ann[:, 1:2]
    bx2 = ann[:, 2:3]
    by2 = ann[:, 3:4]
    blab = ann[:, 4:5]
    anc = anc_ref[0, 0]
    ax1 = anc[0:1, :]
    ay1 = anc[1:2, :]
    ax2 = anc[2:3, :]
    ay2 = anc[3:4, :]

    iw = jnp.maximum(jnp.minimum(ax2, bx2) - jnp.maximum(ax1, bx1), 0.0)
    ih = jnp.maximum(jnp.minimum(ay2, by2) - jnp.maximum(ay1, by1), 0.0)
    inter = iw * ih
    area_a = (ax2 - ax1) * (ay2 - ay1)
    area_b = (bx2 - bx1) * (by2 - by1)
    union = jnp.maximum(area_a + area_b - inter, 1e-8)
    iou = inter / union
    iou = jnp.where(blab != -1.0, iou, -1.0)

    iou_max = jnp.max(iou, axis=0, keepdims=True)
    mi = lax.broadcasted_iota(jnp.int32, (m, bn), 0)
    amax = jnp.min(jnp.where(iou == iou_max, mi, m), axis=0, keepdims=True)
    sel = (mi == amax).astype(jnp.float32)

    gx1 = jnp.sum(sel * bx1, axis=0, keepdims=True)
    gy1 = jnp.sum(sel * by1, axis=0, keepdims=True)
    gx2 = jnp.sum(sel * bx2, axis=0, keepdims=True)
    gy2 = jnp.sum(sel * by2, axis=0, keepdims=True)
    glab = jnp.sum(sel * blab, axis=0, keepdims=True)

    pos = iou_max >= 0.5
    keep = jnp.logical_or(iou_max < 0.4, pos)
    npos_part = jnp.sum(pos.astype(jnp.float32))

    aw = ax2 - ax1
    ah = ay2 - ay1
    acx = ax1 + 0.5 * aw
    acy = ay1 + 0.5 * ah
    gw = gx2 - gx1
    gh = gy2 - gy1
    gcx = gx1 + 0.5 * gw
    gcy = gy1 + 0.5 * gh
    gw = jnp.maximum(gw, 1.0)
    gh = jnp.maximum(gh, 1.0)
    t0 = ((gcx - acx) / aw) / 0.1
    t1 = ((gcy - acy) / ah) / 0.1
    t2 = jnp.log(gw / aw) / 0.2
    t3 = jnp.log(gh / ah) / 0.2
    reg = reg_ref[0, 0]
    d0 = jnp.abs(t0 - reg[0:1, :])
    d1 = jnp.abs(t1 - reg[1:2, :])
    d2 = jnp.abs(t2 - reg[2:3, :])
    d3 = jnp.abs(t3 - reg[3:4, :])

    def smooth_l1(d):
        return jnp.where(d < 1.0 / 9.0, 0.5 * 9.0 * (d * d), d - 0.5 / 9.0)

    rl = smooth_l1(d0) + smooth_l1(d1) + smooth_l1(d2) + smooth_l1(d3)
    reg_part = jnp.sum(jnp.where(pos, rl, 0.0))

    keep_col = jnp.transpose(keep.astype(jnp.float32), (1, 0))
    glab_col = jnp.transpose(glab, (1, 0))

    cls = jnp.clip(cls_ref[0], 0.0001, 1.0 - 0.0001)
    logm = jnp.log(1.0 - cls)
    masked = ((0.75 * (cls * cls)) * logm) * keep_col
    ci = lax.broadcasted_iota(jnp.int32, (bn, c), 1)
    g_col = jnp.sum(jnp.where(ci == glab_col.astype(jnp.int32), cls, 0.0),
                    axis=1, keepdims=True)
    g = jnp.transpose(g_col, (1, 0))
    pos_e = (0.25 * ((1.0 - g) * (1.0 - g))) * (-jnp.log(g))
    neg_e = (0.75 * (g * g)) * (-jnp.log(1.0 - g))
    corr = jnp.where(pos, pos_e - neg_e, 0.0)
    cls_part = jnp.sum(corr) - jnp.sum(masked)

    cls_o[...] += jnp.full(cls_o.shape, cls_part, jnp.float32)
    npos_o[...] += jnp.full(npos_o.shape, npos_part, jnp.float32)
    reg_o[...] += jnp.full(reg_o.shape, reg_part, jnp.float32)


@jax.jit
def kernel(classifications, regressions, anchors, annotations):
    b, n, c = classifications.shape
    m = annotations.shape[1]
    bn = _BN
    nb = n // bn
    anc_t = anchors.reshape(b, nb, bn, 4).transpose(0, 1, 3, 2)
    reg_t = regressions.reshape(b, nb, bn, 4).transpose(0, 1, 3, 2)

    body = functools.partial(_body, bn=bn, m=m, c=c)
    out_sds = jax.ShapeDtypeStruct((b, 1, 128), jnp.float32)
    cls_s, npos, reg_s = pl.pallas_call(
        body,
        grid=(b, nb),
        in_specs=[
            pl.BlockSpec((1, bn, c), lambda bb, ii: (bb, ii, 0)),
            pl.BlockSpec((1, 1, 4, bn), lambda bb, ii: (bb, ii, 0, 0)),
            pl.BlockSpec((1, 1, 4, bn), lambda bb, ii: (bb, ii, 0, 0)),
            pl.BlockSpec((1, m, 5), lambda bb, ii: (bb, 0, 0)),
        ],
        out_specs=[
            pl.BlockSpec((1, 1, 128), lambda bb, ii: (bb, 0, 0)),
            pl.BlockSpec((1, 1, 128), lambda bb, ii: (bb, 0, 0)),
            pl.BlockSpec((1, 1, 128), lambda bb, ii: (bb, 0, 0)),
        ],
        out_shape=[out_sds, out_sds, out_sds],
    )(classifications, reg_t, anc_t, annotations)

    cls_s = cls_s[:, 0, 0]
    npos = npos[:, 0, 0]
    reg_s = reg_s[:, 0, 0]
    cls_tot = jnp.where(npos > 0, cls_s / jnp.maximum(npos, 1.0), 0.0)
    reg_tot = jnp.where(npos > 0, reg_s / jnp.maximum(4.0 * npos, 1.0), 0.0)
    return jnp.mean(cls_tot), jnp.mean(reg_tot)
